# baseline (speedup 1.0000x reference)
import jax
import jax.numpy as jnp
from jax.experimental import pallas as pl
from jax.experimental.pallas import tpu as pltpu

_B, _N, _C = 8, 20000, 91
_NPAD = 20480
_LCHUNK = 128
_SCHUNK = _NPAD // _LCHUNK
_KROWS = 128
_KOUT = 100


def _rowmax_kernel(x_ref, o_ref):
    xt = x_ref[0].T
    o_ref[:, :, :_N] = jnp.max(xt, axis=0)[None, None, :]
    o_ref[:, :, _N:] = jnp.full((1, 1, _NPAD - _N), -jnp.inf, jnp.float32)


def _selrows_kernel(rm_ref, rows_ref, scr_ref):
    scr_ref[...] = rm_ref[...]
    cm0 = jnp.max(rm_ref[...], axis=2)
    lane = jax.lax.broadcasted_iota(jnp.int32, (_B, _LCHUNK), 1)
    sio = jax.lax.broadcasted_iota(jnp.int32, (_B, 1), 0)
    kio = jax.lax.broadcasted_iota(jnp.int32, (_B, _KROWS), 1)
    iota_s = jax.lax.broadcasted_iota(jnp.int32, (1, _SCHUNK), 1)

    def body(i, carry):
        cm, acc = carry
        m = jnp.max(cm, axis=1, keepdims=True)
        l = jnp.min(jnp.where(cm == m, lane, jnp.int32(_LCHUNK)), axis=1,
                    keepdims=True)
        ncm = jnp.zeros((_B, 1), jnp.float32)
        rvec = jnp.zeros((_B, 1), jnp.int32)
        for b in range(_B):
            lb = l[b, 0]
            col = scr_ref[b, pl.ds(lb, 1), :]
            s = jnp.min(jnp.where(col == m[b, 0], iota_s, jnp.int32(_SCHUNK)))
            col2 = jnp.where(iota_s == s, -jnp.inf, col)
            scr_ref[b, pl.ds(lb, 1), :] = col2
            ncm = jnp.where(sio == b, jnp.max(col2), ncm)
            rvec = jnp.where(sio == b, lb * _SCHUNK + s, rvec)
        cm = jnp.where(lane == l, ncm, cm)
        acc = jnp.where(kio == i, rvec, acc)
        return cm, acc

    _, acc = jax.lax.fori_loop(
        0, _KROWS, body, (cm0, jnp.zeros((_B, _KROWS), jnp.int32)))
    rows_ref[...] = acc


def _topk_kernel(cand_ref, rows_ref, scores_ref, q_ref, scr_ref):
    scr_ref[...] = cand_ref[...]
    cm0 = jnp.max(cand_ref[...], axis=2)
    rows = rows_ref[...]
    lane = jax.lax.broadcasted_iota(jnp.int32, (_B, _KROWS), 1)
    sio = jax.lax.broadcasted_iota(jnp.int32, (_B, 1), 0)
    kio = jax.lax.broadcasted_iota(jnp.int32, (_B, _KOUT), 1)
    iota_c = jax.lax.broadcasted_iota(jnp.int32, (1, _C), 1)
    big = jnp.int32(_N)

    def body(i, carry):
        cm, accv, accq = carry
        m = jnp.max(cm, axis=1, keepdims=True)
        win = cm == m
        rmin = jnp.min(jnp.where(win, rows, big), axis=1, keepdims=True)
        l = jnp.min(jnp.where(win & (rows == rmin), lane, jnp.int32(_KROWS)),
                    axis=1, keepdims=True)
        ncm = jnp.zeros((_B, 1), jnp.float32)
        qvec = jnp.zeros((_B, 1), jnp.int32)
        for b in range(_B):
            lb = l[b, 0]
            col = scr_ref[b, pl.ds(lb, 1), :]
            c = jnp.min(jnp.where(col == m[b, 0], iota_c, jnp.int32(_C)))
            col2 = jnp.where(iota_c == c, -jnp.inf, col)
            scr_ref[b, pl.ds(lb, 1), :] = col2
            ncm = jnp.where(sio == b, jnp.max(col2), ncm)
            qvec = jnp.where(sio == b, rmin[b, 0] * _C + c, qvec)
        cm = jnp.where(lane == l, ncm, cm)
        accv = jnp.where(kio == i, m, accv)
        accq = jnp.where(kio == i, qvec, accq)
        return cm, accv, accq

    _, vals, qs = jax.lax.fori_loop(
        0, _KOUT, body,
        (cm0, jnp.zeros((_B, _KOUT), jnp.float32),
         jnp.zeros((_B, _KOUT), jnp.int32)))
    scores_ref[...] = jax.nn.sigmoid(vals)
    q_ref[...] = qs


def _full_spec(*shape):
    return pl.BlockSpec(shape, lambda: tuple(0 for _ in shape))


def kernel(pred_logits, pred_boxes, target_sizes):
    del target_sizes
    rowmax = pl.pallas_call(
        _rowmax_kernel,
        grid=(_B,),
        in_specs=[pl.BlockSpec((1, _N, _C), lambda b: (b, 0, 0))],
        out_specs=pl.BlockSpec((1, 1, _NPAD), lambda b: (b, 0, 0)),
        out_shape=jax.ShapeDtypeStruct((_B, 1, _NPAD), jnp.float32),
    )(pred_logits).reshape(_B, _LCHUNK, _SCHUNK)

    cand_rows = pl.pallas_call(
        _selrows_kernel,
        in_specs=[_full_spec(_B, _LCHUNK, _SCHUNK)],
        out_specs=_full_spec(_B, _KROWS),
        out_shape=jax.ShapeDtypeStruct((_B, _KROWS), jnp.int32),
        scratch_shapes=[pltpu.VMEM((_B, _LCHUNK, _SCHUNK), jnp.float32)],
    )(rowmax)

    cand = jnp.take_along_axis(pred_logits, cand_rows[:, :, None], axis=1)

    scores, q = pl.pallas_call(
        _topk_kernel,
        in_specs=[_full_spec(_B, _KROWS, _C), _full_spec(_B, _KROWS)],
        out_specs=[_full_spec(_B, _KOUT), _full_spec(_B, _KOUT)],
        out_shape=[
            jax.ShapeDtypeStruct((_B, _KOUT), jnp.float32),
            jax.ShapeDtypeStruct((_B, _KOUT), jnp.int32),
        ],
        scratch_shapes=[pltpu.VMEM((_B, _KROWS, _C), jnp.float32)],
    )(cand, cand_rows)

    labels = q % _C
    boxes = jnp.take_along_axis(pred_boxes, (q // _C)[:, :, None], axis=1)
    return scores, labels, boxes


# speedup vs baseline: 2.6078x; 2.6078x over previous
import jax
import jax.numpy as jnp
from jax.experimental import pallas as pl

_B, _N, _C = 8, 20000, 91
_G = 16
_NG = _N // _G
_KROWS = 128
_KOUT = 100


def _extract_topk(x, key, k, width, big):
    lane = jax.lax.broadcasted_iota(jnp.int32, (_B, k), 1)

    def body(i, carry):
        x, accv, accq = carry
        m = jnp.max(x, axis=1, keepdims=True)
        idx = jnp.min(jnp.where(x == m, key, big), axis=1, keepdims=True)
        accv = jnp.where(lane == i, m, accv)
        accq = jnp.where(lane == i, idx, accq)
        x = jnp.where(key == idx, -jnp.inf, x)
        return x, accv, accq

    _, vals, qs = jax.lax.fori_loop(
        0, k, body,
        (x, jnp.zeros((_B, k), jnp.float32), jnp.zeros((_B, k), jnp.int32)))
    return vals, qs


def _rowmax_kernel(x_ref, o_ref):
    xt = x_ref[0].T
    o_ref[...] = jnp.max(xt, axis=0)[None, None, :]


def _topgroups_kernel(rm_ref, grp_ref):
    gm = jnp.max(rm_ref[...].reshape(_B, _NG, _G), axis=2)
    key = jax.lax.broadcasted_iota(jnp.int32, (_B, _NG), 1)
    _, qs = _extract_topk(gm, key, _KROWS, _NG, jnp.int32(_NG))
    grp_ref[...] = qs


def _toprows_kernel(win_ref, key_ref, rows_ref):
    _, qs = _extract_topk(win_ref[...], key_ref[...], _KROWS, _KROWS * _G,
                          jnp.int32(_N))
    rows_ref[...] = qs


def _topk_kernel(cand_ref, key_ref, scores_ref, q_ref):
    vals, qs = _extract_topk(cand_ref[...], key_ref[...], _KOUT, _KROWS * _C,
                             jnp.int32(_N * _C))
    scores_ref[...] = jax.nn.sigmoid(vals)
    q_ref[...] = qs


def _full_spec(*shape):
    return pl.BlockSpec(shape, lambda: tuple(0 for _ in shape))


def kernel(pred_logits, pred_boxes, target_sizes):
    del target_sizes
    rowmax = pl.pallas_call(
        _rowmax_kernel,
        grid=(_B,),
        in_specs=[pl.BlockSpec((1, _N, _C), lambda b: (b, 0, 0))],
        out_specs=pl.BlockSpec((1, 1, _N), lambda b: (b, 0, 0)),
        out_shape=jax.ShapeDtypeStruct((_B, 1, _N), jnp.float32),
    )(pred_logits).reshape(_B, _N)

    top_groups = pl.pallas_call(
        _topgroups_kernel,
        in_specs=[_full_spec(_B, _N)],
        out_specs=_full_spec(_B, _KROWS),
        out_shape=jax.ShapeDtypeStruct((_B, _KROWS), jnp.int32),
    )(rowmax)

    windows = jnp.take_along_axis(
        rowmax.reshape(_B, _NG, _G), top_groups[:, :, None], axis=1)
    win_rows = (top_groups[:, :, None] * _G +
                jnp.arange(_G, dtype=jnp.int32)[None, None, :])

    cand_rows = pl.pallas_call(
        _toprows_kernel,
        in_specs=[_full_spec(_B, _KROWS * _G), _full_spec(_B, _KROWS * _G)],
        out_specs=_full_spec(_B, _KROWS),
        out_shape=jax.ShapeDtypeStruct((_B, _KROWS), jnp.int32),
    )(windows.reshape(_B, _KROWS * _G), win_rows.reshape(_B, _KROWS * _G))

    cand = jnp.take_along_axis(pred_logits, cand_rows[:, :, None], axis=1)
    gflat = (cand_rows[:, :, None] * _C +
             jnp.arange(_C, dtype=jnp.int32)[None, None, :])

    scores, q = pl.pallas_call(
        _topk_kernel,
        in_specs=[_full_spec(_B, _KROWS * _C), _full_spec(_B, _KROWS * _C)],
        out_specs=[_full_spec(_B, _KOUT), _full_spec(_B, _KOUT)],
        out_shape=[
            jax.ShapeDtypeStruct((_B, _KOUT), jnp.float32),
            jax.ShapeDtypeStruct((_B, _KOUT), jnp.int32),
        ],
    )(cand.reshape(_B, _KROWS * _C), gflat.reshape(_B, _KROWS * _C))

    labels = q % _C
    boxes = jnp.take_along_axis(pred_boxes, (q // _C)[:, :, None], axis=1)
    return scores, labels, boxes
